# Initial kernel scaffold; baseline (speedup 1.0000x reference)
import jax
import jax.numpy as jnp
from jax.experimental import pallas as pl


def kernel(x, edge_index, edge_attr, W1, b1, gamma, beta, W2, b2):
    raise NotImplementedError("write your pallas kernel here")


# speedup vs baseline: 5.1073x; 5.1073x over previous
import functools

import jax
import jax.numpy as jnp
from jax import lax
from jax.experimental import pallas as pl
from jax.experimental.pallas import tpu as pltpu
from jax.experimental.pallas import tpu_sc as plsc

N = 10000
E = 160000
EDGE_DIM = 16
NODE_DIM = 256
HIDDEN = 256

NC = 2
NS = 16
NW = NC * NS
CHUNK = 128
NCHUNKS = E // CHUNK
BASE_CHUNKS = NCHUNKS // NW
EXTRA = NCHUNKS - BASE_CHUNKS * NW
MAXC = BASE_CHUNKS + 1
NPAD = 10240
SLAB = NPAD // NS
CROWS = NPAD // EDGE_DIM
CSLAB = CROWS // NS


def _sc_body(edge_hbm, col_hbm, zeros_hbm, rowidx_hbm,
             sums_hbm, cnt_hbm,
             edge_buf, idx2d, hist, idxrow, sum_tab, cnt_tab):
    cid = lax.axis_index("c")
    sid = lax.axis_index("s")
    wid = sid * NC + cid

    pltpu.sync_copy(zeros_hbm, hist)
    pltpu.sync_copy(rowidx_hbm, idxrow)
    pltpu.sync_copy(hist, sum_tab.at[pl.ds(sid * SLAB, SLAB)])
    pltpu.sync_copy(hist.at[pl.ds(0, CSLAB)], cnt_tab.at[pl.ds(sid * CSLAB, CSLAB)])
    plsc.subcore_barrier()

    base = wid * BASE_CHUNKS
    pltpu.sync_copy(edge_hbm.at[pl.ds(base * CHUNK, BASE_CHUNKS * CHUNK)],
                    edge_buf.at[pl.ds(0, BASE_CHUNKS * CHUNK)])
    pltpu.sync_copy(col_hbm.at[pl.ds(base, BASE_CHUNKS)],
                    idx2d.at[pl.ds(0, BASE_CHUNKS)])

    @pl.when(wid < EXTRA)
    def _load_extra():
        xc = NW * BASE_CHUNKS + wid
        pltpu.sync_copy(edge_hbm.at[pl.ds(xc * CHUNK, CHUNK)],
                        edge_buf.at[pl.ds(BASE_CHUNKS * CHUNK, CHUNK)])
        pltpu.sync_copy(col_hbm.at[pl.ds(xc, 1)],
                        idx2d.at[pl.ds(BASE_CHUNKS, 1)])


    ones16 = jnp.ones((16,), jnp.float32)

    def _do_chunk(j):
        pltpu.sync_copy(edge_buf.at[pl.ds(j * CHUNK, CHUNK)],
                        sum_tab.at[idx2d.at[j, 0]], add=True)
        def _cnt(t, _):
            v = idx2d[j, 0, pl.ds(t * 16, 16)]
            plsc.addupdate_scatter(
                hist, [lax.shift_right_logical(v, 4), lax.bitwise_and(v, 15)],
                ones16)
            return 0
        lax.fori_loop(0, CHUNK // 16, _cnt, 0)

    lax.fori_loop(0, BASE_CHUNKS, lambda j, _: (_do_chunk(j), 0)[1], 0)

    @pl.when(wid < EXTRA)
    def _extra_chunk():
        _do_chunk(BASE_CHUNKS)

    def _merge(k, _):
        pltpu.sync_copy(hist.at[pl.ds(k * CHUNK, CHUNK)],
                        cnt_tab.at[idxrow.at[k, 0]], add=True)
        return 0
    lax.fori_loop(0, CROWS // CHUNK, _merge, 0)
    plsc.subcore_barrier()

    pltpu.sync_copy(sum_tab.at[pl.ds(sid * SLAB, SLAB)],
                    sums_hbm.at[cid, pl.ds(sid * SLAB, SLAB)])
    pltpu.sync_copy(cnt_tab.at[pl.ds(sid * CSLAB, CSLAB)],
                    cnt_hbm.at[cid, pl.ds(sid * CSLAB, CSLAB)])


_sc_scatter = functools.partial(
    pl.kernel,
    out_type=(jax.ShapeDtypeStruct((NC, NPAD, EDGE_DIM), jnp.float32),
              jax.ShapeDtypeStruct((NC, CROWS, EDGE_DIM), jnp.float32)),
    mesh=plsc.VectorSubcoreMesh(core_axis_name="c", subcore_axis_name="s",
                                num_cores=NC, num_subcores=NS),
    scratch_types=[
        pltpu.VMEM((MAXC * CHUNK, EDGE_DIM), jnp.float32),
        pltpu.VMEM((MAXC, 1, CHUNK), jnp.int32),
        pltpu.VMEM((CROWS, EDGE_DIM), jnp.float32),
        pltpu.VMEM((CROWS // CHUNK, 1, CHUNK), jnp.int32),
        pltpu.VMEM_SHARED((NPAD, EDGE_DIM), jnp.float32),
        pltpu.VMEM_SHARED((CROWS, EDGE_DIM), jnp.float32),
    ],
    compiler_params=pltpu.CompilerParams(use_tc_tiling_on_sc=False,
                                         needs_layout_passes=False),
)(_sc_body)


ROWS_BLK = 400
GRID = N // ROWS_BLK


def _mlp_body(x_ref, s_ref, c_ref, w1x_ref, w1a_ref, b1_ref, g_ref, be_ref,
              w2_ref, b2_ref, o_ref):
    cnt = jnp.maximum(c_ref[0] + c_ref[1], 1.0)
    agg = (s_ref[0] + s_ref[1]) / cnt
    h = jnp.dot(x_ref[...], w1x_ref[...], preferred_element_type=jnp.float32)
    h = h + jnp.dot(agg, w1a_ref[...], preferred_element_type=jnp.float32)
    h = h + b1_ref[...]
    h = jnp.maximum(h, 0.0)
    mu = jnp.mean(h, axis=1, keepdims=True)
    d = h - mu
    var = jnp.mean(d * d, axis=1, keepdims=True)
    hn = d * lax.rsqrt(var + 1e-5) * g_ref[...] + be_ref[...]
    o_ref[...] = (jnp.dot(hn, w2_ref[...], preferred_element_type=jnp.float32)
                  + b2_ref[...])


def _tc_mlp(x, sums, cnt3, w1x, w1a, b1, gamma, beta, w2, b2):
    full = lambda shape: pl.BlockSpec(shape, lambda i: (0,) * len(shape))
    return pl.pallas_call(
        _mlp_body,
        grid=(GRID,),
        in_specs=[
            pl.BlockSpec((ROWS_BLK, NODE_DIM), lambda i: (i, 0)),
            pl.BlockSpec((NC, ROWS_BLK, EDGE_DIM), lambda i: (0, i, 0)),
            pl.BlockSpec((NC, ROWS_BLK, 1), lambda i: (0, i, 0)),
            full((NODE_DIM, HIDDEN)),
            full((EDGE_DIM, HIDDEN)),
            full((1, HIDDEN)),
            full((1, HIDDEN)),
            full((1, HIDDEN)),
            full((HIDDEN, HIDDEN)),
            full((1, HIDDEN)),
        ],
        out_specs=pl.BlockSpec((ROWS_BLK, HIDDEN), lambda i: (i, 0)),
        out_shape=jax.ShapeDtypeStruct((N, HIDDEN), jnp.float32),
        compiler_params=pltpu.CompilerParams(
            dimension_semantics=("arbitrary",)),
    )(x, sums, cnt3, w1x, w1a, b1, gamma, beta, w2, b2)


def kernel(x, edge_index, edge_attr, W1, b1, gamma, beta, W2, b2):
    col3d = edge_index[1].astype(jnp.int32).reshape(NCHUNKS, 1, CHUNK)
    zeros_c = jnp.zeros((CROWS, EDGE_DIM), jnp.float32)
    rowidx_c = jnp.arange(CROWS, dtype=jnp.int32).reshape(-1, 1, CHUNK)
    sums, cnt = _sc_scatter(edge_attr, col3d, zeros_c, rowidx_c)
    cnt3 = cnt.reshape(NC, NPAD, 1)
    return _tc_mlp(x, sums, cnt3, W1[:NODE_DIM], W1[NODE_DIM:],
                   b1.reshape(1, HIDDEN), gamma.reshape(1, HIDDEN),
                   beta.reshape(1, HIDDEN), W2, b2.reshape(1, HIDDEN))
